# baseline (speedup 1.0000x reference)
import functools

import jax
import jax.numpy as jnp
from jax import lax
from jax.experimental import pallas as pl
from jax.experimental.pallas import tpu as pltpu
from jax.experimental.pallas import tpu_sc as plsc

N = 10000
E = 320000
D = 128
C = 16

NC = 2
NS = 16
NW = NC * NS
K = 128
STEPS = 80
EP = NW * STEPS * K
NP_ = 10240
ROWS_PER_TILE = NP_ // NS


def _onehot_body(logits_ref, p_ref):
    lg = logits_ref[...]
    m = jnp.max(lg, axis=1, keepdims=True)
    col = lax.broadcasted_iota(jnp.int32, lg.shape, 1)
    idx = jnp.min(jnp.where(lg == m, col, C), axis=1, keepdims=True)
    p_ref[...] = (col == idx).astype(jnp.float32)


def _onehot_pred(logits):
    return pl.pallas_call(
        _onehot_body,
        out_shape=jax.ShapeDtypeStruct((N, C), jnp.float32),
    )(logits)


def _sc_body(src_hbm, dst_hbm, p_hbm, h_hbm, zc_hbm, zd_hbm,
             cnts_out, agg_out,
             srcb, dstb, oh_v, row_v, cnts_sh, agg_sh,
             idx_sem, goh_sem, grow_sem, soh_sem, srow_sem):
    c = lax.axis_index("c")
    s = lax.axis_index("s")
    wid = s * NC + c

    r0 = s * ROWS_PER_TILE
    pltpu.sync_copy(zc_hbm.at[pl.ds(r0, ROWS_PER_TILE)],
                    cnts_sh.at[pl.ds(r0, ROWS_PER_TILE)])
    pltpu.sync_copy(zd_hbm.at[pl.ds(r0, ROWS_PER_TILE)],
                    agg_sh.at[pl.ds(r0, ROWS_PER_TILE)])
    plsc.subcore_barrier()

    def issue_idx(i, b):
        iw = lax.rem(i, STEPS)
        pltpu.async_copy(src_hbm.at[wid, iw], srcb.at[b], idx_sem)
        pltpu.async_copy(dst_hbm.at[wid, iw], dstb.at[b], idx_sem)

    def wait_idx(b):
        pltpu.make_async_copy(src_hbm.at[wid, 0], srcb.at[b],
                              idx_sem).wait()
        pltpu.make_async_copy(dst_hbm.at[wid, 0], dstb.at[b],
                              idx_sem).wait()

    def issue_gathers(b):
        pltpu.async_copy(p_hbm.at[srcb.at[b]], oh_v.at[b], goh_sem)
        pltpu.async_copy(h_hbm.at[srcb.at[b]], row_v.at[b], grow_sem)

    def wait_gathers(b):
        pltpu.make_async_copy(p_hbm.at[srcb.at[b]], oh_v.at[b],
                              goh_sem).wait()
        pltpu.make_async_copy(h_hbm.at[srcb.at[b]], row_v.at[b],
                              grow_sem).wait()

    def issue_scatters(b):
        pltpu.async_copy(oh_v.at[b], cnts_sh.at[dstb.at[b]], soh_sem,
                         add=True)
        pltpu.async_copy(row_v.at[b], agg_sh.at[dstb.at[b]], srow_sem,
                         add=True)

    def wait_scatters(b):
        pltpu.make_async_copy(oh_v.at[b], cnts_sh.at[pl.ds(0, K)],
                              soh_sem).wait()
        pltpu.make_async_copy(row_v.at[b], agg_sh.at[pl.ds(0, K)],
                              srow_sem).wait()

    issue_idx(0, 0)
    wait_idx(0)
    issue_gathers(0)
    issue_idx(1, 1)

    def step(g, carry):
        i0 = 2 * g
        wait_gathers(0)
        issue_scatters(0)
        issue_idx(i0 + 2, 0)
        wait_idx(1)
        issue_gathers(1)
        wait_gathers(1)
        issue_scatters(1)
        issue_idx(i0 + 3, 1)
        wait_scatters(0)
        wait_idx(0)
        issue_gathers(0)
        wait_scatters(1)
        return carry

    lax.fori_loop(0, STEPS // 2 - 1, step, 0)
    wait_gathers(0)
    issue_scatters(0)
    wait_idx(1)
    issue_gathers(1)
    wait_gathers(1)
    issue_scatters(1)
    wait_scatters(0)
    wait_scatters(1)
    plsc.subcore_barrier()

    pltpu.sync_copy(cnts_sh.at[pl.ds(r0, ROWS_PER_TILE)],
                    cnts_out.at[c, pl.ds(r0, ROWS_PER_TILE)])
    pltpu.sync_copy(agg_sh.at[pl.ds(r0, ROWS_PER_TILE)],
                    agg_out.at[c, pl.ds(r0, ROWS_PER_TILE)])


def _sc_aggregate(src, dst, p, h, zc, zd):
    mesh = plsc.VectorSubcoreMesh(core_axis_name="c", subcore_axis_name="s")
    f = pl.kernel(
        _sc_body,
        out_type=(
            jax.ShapeDtypeStruct((NC, NP_, C), jnp.float32),
            jax.ShapeDtypeStruct((NC, NP_, D), jnp.float32),
        ),
        mesh=mesh,
        scratch_types=[
            pltpu.VMEM((2, K), jnp.int32),
            pltpu.VMEM((2, K), jnp.int32),
            pltpu.VMEM((2, K, C), jnp.float32),
            pltpu.VMEM((2, K, D), jnp.float32),
            pltpu.VMEM_SHARED((NP_, C), jnp.float32),
            pltpu.VMEM_SHARED((NP_, D), jnp.float32),
            pltpu.SemaphoreType.DMA,
            pltpu.SemaphoreType.DMA,
            pltpu.SemaphoreType.DMA,
            pltpu.SemaphoreType.DMA,
            pltpu.SemaphoreType.DMA,
        ],
        compiler_params=pltpu.CompilerParams(use_tc_tiling_on_sc=False),
    )
    return f(src.reshape(NW, STEPS, K), dst.reshape(NW, STEPS, K),
             p, h, zc, zd)


def _gate_body(cnts2_ref, p_ref, oldz_ref, t1_ref, t2_ref, z_ref, gate_ref):
    cnts = cnts2_ref[0] + cnts2_ref[1]
    p = p_ref[...]
    f1 = jnp.sum(cnts * p, axis=1, keepdims=True)
    cc = jnp.maximum(cnts, 1e-5)
    f2 = -jnp.sum(cc * jnp.log(cc), axis=1, keepdims=True)

    def _ln(x):
        mu = jnp.mean(x)
        var = jnp.mean((x - mu) ** 2)
        return (x - mu) / jnp.sqrt(var + 1e-5)

    nf1 = _ln(f1)
    nf2 = _ln(f2)
    t1 = t1_ref[0, 0]
    t2 = t2_ref[0, 0]
    z = jax.nn.sigmoid(t1 - nf1) * jax.nn.sigmoid(t2 - nf2)
    z_ref[...] = z
    gate_ref[...] = jnp.minimum(oldz_ref[...], z)


def _gates(cnts2, p, old_z, tau_1, tau_2):
    return pl.pallas_call(
        _gate_body,
        out_shape=(
            jax.ShapeDtypeStruct((N, 1), jnp.float32),
            jax.ShapeDtypeStruct((N, 1), jnp.float32),
        ),
    )(cnts2, p, old_z, tau_1, tau_2)


BLK = 1000


def _update_body(h_ref, a0_ref, a1_ref, gate_ref, out_ref):
    agg = jax.nn.relu(a0_ref[...] + a1_ref[...])
    out_ref[...] = h_ref[...] + gate_ref[...] * agg


def _update(h, a0, a1, gate):
    grid = (N // BLK,)
    spec = pl.BlockSpec((BLK, D), lambda i: (i, 0))
    gspec = pl.BlockSpec((BLK, 1), lambda i: (i, 0))
    return pl.pallas_call(
        _update_body,
        grid=grid,
        in_specs=[spec, spec, spec, gspec],
        out_specs=spec,
        out_shape=jax.ShapeDtypeStruct((N, D), jnp.float32),
    )(h, a0, a1, gate)


def kernel(h, logits, old_z, edge_index, tau_1, tau_2):
    src = edge_index[0].astype(jnp.int32)
    dst = edge_index[1].astype(jnp.int32)
    pad = EP - E
    src = jnp.concatenate([src, jnp.zeros((pad,), jnp.int32)])
    dst = jnp.concatenate(
        [dst, N + (jnp.arange(pad, dtype=jnp.int32) % (NP_ - N))])

    p = _onehot_pred(logits)

    zc = jnp.zeros((NP_, C), jnp.float32)
    zd = jnp.zeros((NP_, D), jnp.float32)
    cnts2, agg2 = _sc_aggregate(src, dst, p, h, zc, zd)
    cnts2 = cnts2[:, :N, :]
    agg2 = agg2[:, :N, :]

    z, gate = _gates(cnts2, p, old_z.reshape(N, 1),
                     tau_1.reshape(1, 1), tau_2.reshape(1, 1))
    new_h = _update(h, agg2[0], agg2[1], gate)
    return (new_h, z.reshape(N))


# speedup vs baseline: 1.7311x; 1.7311x over previous
import functools

import jax
import jax.numpy as jnp
from jax import lax
from jax.experimental import pallas as pl
from jax.experimental.pallas import tpu as pltpu
from jax.experimental.pallas import tpu_sc as plsc

N = 10000
E = 320000
D = 128
C = 16

NC = 2
NS = 16
NW = NC * NS
K = 80
STEPS = 126
EP = NW * STEPS * K
NP_ = 10240
ROWS_PER_TILE = NP_ // NS


def _onehot_body(logits_ref, p_ref):
    lg = logits_ref[...]
    m = jnp.max(lg, axis=1, keepdims=True)
    col = lax.broadcasted_iota(jnp.int32, lg.shape, 1)
    idx = jnp.min(jnp.where(lg == m, col, C), axis=1, keepdims=True)
    p_ref[...] = (col == idx).astype(jnp.float32)


def _onehot_pred(logits):
    return pl.pallas_call(
        _onehot_body,
        out_shape=jax.ShapeDtypeStruct((N, C), jnp.float32),
    )(logits)


def _sc_body(src_hbm, dst_hbm, p_hbm, h_hbm, zc_hbm, zd_hbm,
             cnts_out, agg_out,
             srcb, dstb, oh_v, row_v, cnts_sh, agg_sh,
             idx_sem, goh_sem, grow_sem, soh_sem, srow_sem):
    c = lax.axis_index("c")
    s = lax.axis_index("s")
    wid = s * NC + c

    r0 = s * ROWS_PER_TILE
    pltpu.sync_copy(zc_hbm.at[pl.ds(r0, ROWS_PER_TILE)],
                    cnts_sh.at[pl.ds(r0, ROWS_PER_TILE)])
    pltpu.sync_copy(zd_hbm.at[pl.ds(r0, ROWS_PER_TILE)],
                    agg_sh.at[pl.ds(r0, ROWS_PER_TILE)])
    plsc.subcore_barrier()

    def issue_idx(i, b):
        iw = lax.rem(i, STEPS)
        pltpu.async_copy(src_hbm.at[wid, iw], srcb.at[b], idx_sem)
        pltpu.async_copy(dst_hbm.at[wid, iw], dstb.at[b], idx_sem)

    def wait_idx(b):
        pltpu.make_async_copy(src_hbm.at[wid, 0], srcb.at[b],
                              idx_sem).wait()
        pltpu.make_async_copy(dst_hbm.at[wid, 0], dstb.at[b],
                              idx_sem).wait()

    def issue_gathers(b):
        pltpu.async_copy(p_hbm.at[srcb.at[b]], oh_v.at[b], goh_sem)
        pltpu.async_copy(h_hbm.at[srcb.at[b]], row_v.at[b], grow_sem)

    def wait_gathers(b):
        pltpu.make_async_copy(p_hbm.at[srcb.at[b]], oh_v.at[b],
                              goh_sem).wait()
        pltpu.make_async_copy(h_hbm.at[srcb.at[b]], row_v.at[b],
                              grow_sem).wait()

    def issue_scatters(b):
        pltpu.async_copy(oh_v.at[b], cnts_sh.at[dstb.at[b]], soh_sem,
                         add=True)
        pltpu.async_copy(row_v.at[b], agg_sh.at[dstb.at[b]], srow_sem,
                         add=True)

    def wait_scatters(b):
        pltpu.make_async_copy(oh_v.at[b], cnts_sh.at[pl.ds(0, K)],
                              soh_sem).wait()
        pltpu.make_async_copy(row_v.at[b], agg_sh.at[pl.ds(0, K)],
                              srow_sem).wait()

    def substep(i_next, b_free, b_cur):
        wait_scatters(b_free)
        issue_idx(i_next, b_free)
        wait_gathers(b_cur)
        issue_scatters(b_cur)
        wait_idx(b_free)
        issue_gathers(b_free)

    issue_idx(0, 0)
    issue_idx(1, 1)
    wait_idx(0)
    issue_gathers(0)
    wait_idx(1)
    issue_gathers(1)
    issue_idx(2, 2)
    wait_gathers(0)
    issue_scatters(0)
    wait_idx(2)
    issue_gathers(2)

    def step(g, carry):
        i0 = 3 * g
        substep(i0 + 3, 0, 1)
        substep(i0 + 4, 1, 2)
        substep(i0 + 5, 2, 0)
        return carry

    lax.fori_loop(0, STEPS // 3 - 1, step, 0)
    substep(STEPS, 0, 1)
    substep(STEPS + 1, 1, 2)
    wait_scatters(2)
    wait_gathers(0)
    wait_gathers(1)
    plsc.subcore_barrier()

    pltpu.sync_copy(cnts_sh.at[pl.ds(r0, ROWS_PER_TILE)],
                    cnts_out.at[c, pl.ds(r0, ROWS_PER_TILE)])
    pltpu.sync_copy(agg_sh.at[pl.ds(r0, ROWS_PER_TILE)],
                    agg_out.at[c, pl.ds(r0, ROWS_PER_TILE)])


def _sc_aggregate(src, dst, p, h, zc, zd):
    mesh = plsc.VectorSubcoreMesh(core_axis_name="c", subcore_axis_name="s")
    f = pl.kernel(
        _sc_body,
        out_type=(
            jax.ShapeDtypeStruct((NC, NP_, C), jnp.float32),
            jax.ShapeDtypeStruct((NC, NP_, D), jnp.float32),
        ),
        mesh=mesh,
        scratch_types=[
            pltpu.VMEM((3, K), jnp.int32),
            pltpu.VMEM((3, K), jnp.int32),
            pltpu.VMEM((3, K, C), jnp.float32),
            pltpu.VMEM((3, K, D), jnp.float32),
            pltpu.VMEM_SHARED((NP_, C), jnp.float32),
            pltpu.VMEM_SHARED((NP_, D), jnp.float32),
            pltpu.SemaphoreType.DMA,
            pltpu.SemaphoreType.DMA,
            pltpu.SemaphoreType.DMA,
            pltpu.SemaphoreType.DMA,
            pltpu.SemaphoreType.DMA,
        ],
        compiler_params=pltpu.CompilerParams(use_tc_tiling_on_sc=False),
    )
    return f(src.reshape(NW, STEPS, K), dst.reshape(NW, STEPS, K),
             p, h, zc, zd)


def _gate_body(cnts2_ref, p_ref, oldz_ref, t1_ref, t2_ref, z_ref, gate_ref):
    cnts = cnts2_ref[0] + cnts2_ref[1]
    p = p_ref[...]
    f1 = jnp.sum(cnts * p, axis=1, keepdims=True)
    cc = jnp.maximum(cnts, 1e-5)
    f2 = -jnp.sum(cc * jnp.log(cc), axis=1, keepdims=True)

    def _ln(x):
        mu = jnp.mean(x)
        var = jnp.mean((x - mu) ** 2)
        return (x - mu) / jnp.sqrt(var + 1e-5)

    nf1 = _ln(f1)
    nf2 = _ln(f2)
    t1 = t1_ref[0, 0]
    t2 = t2_ref[0, 0]
    z = jax.nn.sigmoid(t1 - nf1) * jax.nn.sigmoid(t2 - nf2)
    z_ref[...] = z
    gate_ref[...] = jnp.minimum(oldz_ref[...], z)


def _gates(cnts2, p, old_z, tau_1, tau_2):
    return pl.pallas_call(
        _gate_body,
        out_shape=(
            jax.ShapeDtypeStruct((N, 1), jnp.float32),
            jax.ShapeDtypeStruct((N, 1), jnp.float32),
        ),
    )(cnts2, p, old_z, tau_1, tau_2)


BLK = 1000


def _update_body(h_ref, a0_ref, a1_ref, gate_ref, out_ref):
    agg = jax.nn.relu(a0_ref[...] + a1_ref[...])
    out_ref[...] = h_ref[...] + gate_ref[...] * agg


def _update(h, a0, a1, gate):
    grid = (N // BLK,)
    spec = pl.BlockSpec((BLK, D), lambda i: (i, 0))
    gspec = pl.BlockSpec((BLK, 1), lambda i: (i, 0))
    return pl.pallas_call(
        _update_body,
        grid=grid,
        in_specs=[spec, spec, spec, gspec],
        out_specs=spec,
        out_shape=jax.ShapeDtypeStruct((N, D), jnp.float32),
    )(h, a0, a1, gate)


def kernel(h, logits, old_z, edge_index, tau_1, tau_2):
    src = edge_index[0].astype(jnp.int32)
    dst = edge_index[1].astype(jnp.int32)
    pad = EP - E
    src = jnp.concatenate([src, jnp.zeros((pad,), jnp.int32)])
    dst = jnp.concatenate(
        [dst, N + (jnp.arange(pad, dtype=jnp.int32) % (NP_ - N))])

    p = _onehot_pred(logits)

    zc = jnp.zeros((NP_, C), jnp.float32)
    zd = jnp.zeros((NP_, D), jnp.float32)
    cnts2, agg2 = _sc_aggregate(src, dst, p, h, zc, zd)
    cnts2 = cnts2[:, :N, :]
    agg2 = agg2[:, :N, :]

    z, gate = _gates(cnts2, p, old_z.reshape(N, 1),
                     tau_1.reshape(1, 1), tau_2.reshape(1, 1))
    new_h = _update(h, agg2[0], agg2[1], gate)
    return (new_h, z.reshape(N))


# speedup vs baseline: 2.3292x; 1.3455x over previous
import functools

import jax
import jax.numpy as jnp
from jax import lax
from jax.experimental import pallas as pl
from jax.experimental.pallas import tpu as pltpu
from jax.experimental.pallas import tpu_sc as plsc

N = 10000
E = 320000
D = 128
C = 16

NC = 2
NS = 16
NW = NC * NS
K = 80
STEPS = 125
EP = NW * STEPS * K
NP_ = 10240
ROWS_PER_TILE = NP_ // NS


def _onehot_body(logits_ref, p_ref):
    lg = logits_ref[...]
    m = jnp.max(lg, axis=1, keepdims=True)
    col = lax.broadcasted_iota(jnp.int32, lg.shape, 1)
    idx = jnp.min(jnp.where(lg == m, col, C), axis=1, keepdims=True)
    p_ref[...] = (col == idx).astype(jnp.float32)


def _onehot_pred(logits):
    return pl.pallas_call(
        _onehot_body,
        out_shape=jax.ShapeDtypeStruct((N, C), jnp.float32),
    )(logits)


def _sc_body(src_hbm, dst_hbm, p_hbm, h_hbm, zc_hbm, zd_hbm,
             cnts_out, agg_out,
             srcb, dst_v, oh_v, row_v, cnts_sh, agg_sh,
             idx_sem, goh_sem, grow_sem, soh_sem, srow_sem):
    c = lax.axis_index("c")
    s = lax.axis_index("s")
    wid = s * NC + c

    r0 = s * ROWS_PER_TILE
    pltpu.sync_copy(zc_hbm.at[pl.ds(r0, ROWS_PER_TILE)],
                    cnts_sh.at[pl.ds(r0, ROWS_PER_TILE)])
    pltpu.sync_copy(zd_hbm.at[pl.ds(r0, ROWS_PER_TILE)],
                    agg_sh.at[pl.ds(r0, ROWS_PER_TILE)])
    plsc.subcore_barrier()

    pltpu.sync_copy(dst_hbm.at[wid], dst_v)

    def issue_srcidx(i, b):
        pltpu.async_copy(src_hbm.at[wid, lax.rem(i, STEPS)], srcb.at[b],
                         idx_sem)

    def wait_srcidx(b):
        pltpu.make_async_copy(src_hbm.at[wid, 0], srcb.at[b],
                              idx_sem).wait()

    def issue_gathers(b):
        pltpu.async_copy(p_hbm.at[srcb.at[b]], oh_v.at[b], goh_sem)
        pltpu.async_copy(h_hbm.at[srcb.at[b]], row_v.at[b], grow_sem)

    def wait_gathers(b):
        pltpu.make_async_copy(p_hbm.at[srcb.at[b]], oh_v.at[b],
                              goh_sem).wait()
        pltpu.make_async_copy(h_hbm.at[srcb.at[b]], row_v.at[b],
                              grow_sem).wait()

    def issue_scatters(i, b):
        pltpu.async_copy(oh_v.at[b], cnts_sh.at[dst_v.at[i]], soh_sem,
                         add=True)
        pltpu.async_copy(row_v.at[b], agg_sh.at[dst_v.at[i]], srow_sem,
                         add=True)

    def wait_scatters(b):
        pltpu.make_async_copy(oh_v.at[b], cnts_sh.at[pl.ds(0, K)],
                              soh_sem).wait()
        pltpu.make_async_copy(row_v.at[b], agg_sh.at[pl.ds(0, K)],
                              srow_sem).wait()

    issue_srcidx(0, 0)
    wait_srcidx(0)
    issue_gathers(0)
    issue_srcidx(1, 1)

    def step(g, carry):
        i0 = 2 * g
        i1 = i0 + 1
        wait_srcidx(1)
        issue_gathers(1)
        wait_gathers(0)
        issue_scatters(i0, 0)
        issue_srcidx(i0 + 2, 0)
        wait_gathers(1)
        issue_scatters(i1, 1)
        issue_srcidx(i1 + 2, 1)
        wait_scatters(0)
        wait_srcidx(0)
        issue_gathers(0)
        wait_scatters(1)
        return carry

    lax.fori_loop(0, (STEPS - 1) // 2, step, 0)
    wait_gathers(0)
    issue_scatters(STEPS - 1, 0)
    wait_scatters(0)
    wait_srcidx(1)
    plsc.subcore_barrier()

    pltpu.sync_copy(cnts_sh.at[pl.ds(r0, ROWS_PER_TILE)],
                    cnts_out.at[c, pl.ds(r0, ROWS_PER_TILE)])
    pltpu.sync_copy(agg_sh.at[pl.ds(r0, ROWS_PER_TILE)],
                    agg_out.at[c, pl.ds(r0, ROWS_PER_TILE)])


def _sc_aggregate(src, dst, p, h, zc, zd):
    mesh = plsc.VectorSubcoreMesh(core_axis_name="c", subcore_axis_name="s")
    f = pl.kernel(
        _sc_body,
        out_type=(
            jax.ShapeDtypeStruct((NC, NP_, C), jnp.float32),
            jax.ShapeDtypeStruct((NC, NP_, D), jnp.float32),
        ),
        mesh=mesh,
        scratch_types=[
            pltpu.VMEM((2, K), jnp.int32),
            pltpu.VMEM((STEPS, K), jnp.int32),
            pltpu.VMEM((2, K, C), jnp.float32),
            pltpu.VMEM((2, K, D), jnp.float32),
            pltpu.VMEM_SHARED((NP_, C), jnp.float32),
            pltpu.VMEM_SHARED((NP_, D), jnp.float32),
            pltpu.SemaphoreType.DMA,
            pltpu.SemaphoreType.DMA,
            pltpu.SemaphoreType.DMA,
            pltpu.SemaphoreType.DMA,
            pltpu.SemaphoreType.DMA,
        ],
        compiler_params=pltpu.CompilerParams(use_tc_tiling_on_sc=False),
    )
    return f(src.reshape(NW, STEPS, K), dst.reshape(NW, STEPS, K),
             p, h, zc, zd)


def _gate_body(cnts2_ref, p_ref, oldz_ref, t1_ref, t2_ref, z_ref, gate_ref):
    cnts = cnts2_ref[0] + cnts2_ref[1]
    p = p_ref[...]
    f1 = jnp.sum(cnts * p, axis=1, keepdims=True)
    cc = jnp.maximum(cnts, 1e-5)
    f2 = -jnp.sum(cc * jnp.log(cc), axis=1, keepdims=True)

    def _ln(x):
        mu = jnp.mean(x)
        var = jnp.mean((x - mu) ** 2)
        return (x - mu) / jnp.sqrt(var + 1e-5)

    nf1 = _ln(f1)
    nf2 = _ln(f2)
    t1 = t1_ref[0, 0]
    t2 = t2_ref[0, 0]
    z = jax.nn.sigmoid(t1 - nf1) * jax.nn.sigmoid(t2 - nf2)
    z_ref[...] = z
    gate_ref[...] = jnp.minimum(oldz_ref[...], z)


def _gates(cnts2, p, old_z, tau_1, tau_2):
    return pl.pallas_call(
        _gate_body,
        out_shape=(
            jax.ShapeDtypeStruct((N, 1), jnp.float32),
            jax.ShapeDtypeStruct((N, 1), jnp.float32),
        ),
    )(cnts2, p, old_z, tau_1, tau_2)


BLK = 1000


def _update_body(h_ref, a0_ref, a1_ref, gate_ref, out_ref):
    agg = jax.nn.relu(a0_ref[...] + a1_ref[...])
    out_ref[...] = h_ref[...] + gate_ref[...] * agg


def _update(h, a0, a1, gate):
    grid = (N // BLK,)
    spec = pl.BlockSpec((BLK, D), lambda i: (i, 0))
    gspec = pl.BlockSpec((BLK, 1), lambda i: (i, 0))
    return pl.pallas_call(
        _update_body,
        grid=grid,
        in_specs=[spec, spec, spec, gspec],
        out_specs=spec,
        out_shape=jax.ShapeDtypeStruct((N, D), jnp.float32),
    )(h, a0, a1, gate)


def kernel(h, logits, old_z, edge_index, tau_1, tau_2):
    src = edge_index[0].astype(jnp.int32)
    dst = edge_index[1].astype(jnp.int32)

    p = _onehot_pred(logits)

    zc = jnp.zeros((NP_, C), jnp.float32)
    zd = jnp.zeros((NP_, D), jnp.float32)
    cnts2, agg2 = _sc_aggregate(src, dst, p, h, zc, zd)
    cnts2 = cnts2[:, :N, :]
    agg2 = agg2[:, :N, :]

    z, gate = _gates(cnts2, p, old_z.reshape(N, 1),
                     tau_1.reshape(1, 1), tau_2.reshape(1, 1))
    new_h = _update(h, agg2[0], agg2[1], gate)
    return (new_h, z.reshape(N))


# speedup vs baseline: 2.4639x; 1.0578x over previous
import functools

import jax
import jax.numpy as jnp
from jax import lax
from jax.experimental import pallas as pl
from jax.experimental.pallas import tpu as pltpu
from jax.experimental.pallas import tpu_sc as plsc

N = 10000
E = 320000
D = 128
C = 16

NC = 2
NS = 16
NW = NC * NS
K = 80
STEPS = 125
EP = NW * STEPS * K
NP_ = 10240
ROWS_PER_TILE = NP_ // NS


def _onehot_body(logits_ref, p_ref):
    lg = logits_ref[...]
    m = jnp.max(lg, axis=1, keepdims=True)
    col = lax.broadcasted_iota(jnp.int32, lg.shape, 1)
    idx = jnp.min(jnp.where(lg == m, col, C), axis=1, keepdims=True)
    p_ref[...] = (col == idx).astype(jnp.float32)


def _onehot_pred(logits):
    return pl.pallas_call(
        _onehot_body,
        out_shape=jax.ShapeDtypeStruct((N, C), jnp.float32),
    )(logits)


def _sc_body(src_hbm, dst_hbm, p_hbm, h_hbm, zc_hbm, zd_hbm,
             cnts_out, agg_out,
             srcb, dst_v, oh_v, row_v, cnts_sh, agg_sh,
             idx_sem, goh_sem, grow_sem, soh_sem, srow_sem):
    c = lax.axis_index("c")
    s = lax.axis_index("s")
    wid = s * NC + c

    r0 = s * ROWS_PER_TILE
    pltpu.sync_copy(zc_hbm.at[pl.ds(r0, ROWS_PER_TILE)],
                    cnts_sh.at[pl.ds(r0, ROWS_PER_TILE)])
    pltpu.sync_copy(zd_hbm.at[pl.ds(r0, ROWS_PER_TILE)],
                    agg_sh.at[pl.ds(r0, ROWS_PER_TILE)])
    plsc.subcore_barrier()

    pltpu.sync_copy(dst_hbm.at[wid], dst_v)

    def issue_srcidx(i, b):
        pltpu.async_copy(src_hbm.at[wid, lax.rem(i, STEPS)], srcb.at[b],
                         idx_sem)

    def wait_srcidx(b):
        pltpu.make_async_copy(src_hbm.at[wid, 0], srcb.at[b],
                              idx_sem).wait()

    def issue_gathers(b):
        pltpu.async_copy(p_hbm.at[srcb.at[b]], oh_v.at[b], goh_sem)
        pltpu.async_copy(h_hbm.at[srcb.at[b]], row_v.at[b], grow_sem)

    def wait_gathers(b):
        pltpu.make_async_copy(p_hbm.at[srcb.at[b]], oh_v.at[b],
                              goh_sem).wait()
        pltpu.make_async_copy(h_hbm.at[srcb.at[b]], row_v.at[b],
                              grow_sem).wait()

    def issue_scatters(i, b):
        pltpu.async_copy(oh_v.at[b], cnts_sh.at[dst_v.at[i]], soh_sem,
                         add=True)
        pltpu.async_copy(row_v.at[b], agg_sh.at[dst_v.at[i]], srow_sem,
                         add=True)

    def wait_scatters(b):
        pltpu.make_async_copy(oh_v.at[b], cnts_sh.at[pl.ds(0, K)],
                              soh_sem).wait()
        pltpu.make_async_copy(row_v.at[b], agg_sh.at[pl.ds(0, K)],
                              srow_sem).wait()

    issue_srcidx(0, 0)
    wait_srcidx(0)
    issue_gathers(0)
    issue_srcidx(1, 1)

    def step(g, carry):
        i0 = 2 * g
        i1 = i0 + 1
        wait_srcidx(1)
        issue_gathers(1)
        wait_gathers(0)
        issue_scatters(i0, 0)
        issue_srcidx(i0 + 2, 0)
        wait_gathers(1)
        issue_scatters(i1, 1)
        issue_srcidx(i1 + 2, 1)
        wait_scatters(0)
        wait_srcidx(0)
        issue_gathers(0)
        wait_scatters(1)
        return carry

    lax.fori_loop(0, (STEPS - 1) // 2, step, 0)
    wait_gathers(0)
    issue_scatters(STEPS - 1, 0)
    wait_scatters(0)
    wait_srcidx(1)
    plsc.subcore_barrier()

    pltpu.sync_copy(cnts_sh.at[pl.ds(r0, ROWS_PER_TILE)],
                    cnts_out.at[c, pl.ds(r0, ROWS_PER_TILE)])
    pltpu.sync_copy(agg_sh.at[pl.ds(r0, ROWS_PER_TILE)],
                    agg_out.at[c, pl.ds(r0, ROWS_PER_TILE)])


def _sc_aggregate(src, dst, p, h, zc, zd):
    mesh = plsc.VectorSubcoreMesh(core_axis_name="c", subcore_axis_name="s")
    f = pl.kernel(
        _sc_body,
        out_type=(
            jax.ShapeDtypeStruct((NC, NP_, C), jnp.float32),
            jax.ShapeDtypeStruct((NC, NP_, D), jnp.float32),
        ),
        mesh=mesh,
        scratch_types=[
            pltpu.VMEM((2, K), jnp.int32),
            pltpu.VMEM((STEPS, K), jnp.int32),
            pltpu.VMEM((2, K, C), jnp.float32),
            pltpu.VMEM((2, K, D), jnp.float32),
            pltpu.VMEM_SHARED((NP_, C), jnp.float32),
            pltpu.VMEM_SHARED((NP_, D), jnp.float32),
            pltpu.SemaphoreType.DMA,
            pltpu.SemaphoreType.DMA,
            pltpu.SemaphoreType.DMA,
            pltpu.SemaphoreType.DMA,
            pltpu.SemaphoreType.DMA,
        ],
        compiler_params=pltpu.CompilerParams(use_tc_tiling_on_sc=False),
    )
    return f(src.reshape(NW, STEPS, K), dst.reshape(NW, STEPS, K),
             p, h, zc, zd)


def _gate_body(cnts2_ref, p_ref, oldz_ref, t1_ref, t2_ref, z_ref, gate_ref):
    cnts = cnts2_ref[0, :N, :] + cnts2_ref[1, :N, :]
    p = p_ref[...]
    f1 = jnp.sum(cnts * p, axis=1, keepdims=True)
    cc = jnp.maximum(cnts, 1e-5)
    f2 = -jnp.sum(cc * jnp.log(cc), axis=1, keepdims=True)

    def _ln(x):
        mu = jnp.mean(x)
        var = jnp.mean((x - mu) ** 2)
        return (x - mu) / jnp.sqrt(var + 1e-5)

    nf1 = _ln(f1)
    nf2 = _ln(f2)
    t1 = t1_ref[0, 0]
    t2 = t2_ref[0, 0]
    z = jax.nn.sigmoid(t1 - nf1) * jax.nn.sigmoid(t2 - nf2)
    z_ref[...] = z
    gate_ref[...] = jnp.minimum(oldz_ref[...], z)


def _gates(cnts2, p, old_z, tau_1, tau_2):
    return pl.pallas_call(
        _gate_body,
        out_shape=(
            jax.ShapeDtypeStruct((N, 1), jnp.float32),
            jax.ShapeDtypeStruct((N, 1), jnp.float32),
        ),
    )(cnts2, p, old_z, tau_1, tau_2)


BLK = 1000


def _update_body(h_ref, a0_ref, a1_ref, gate_ref, out_ref):
    agg = jax.nn.relu(a0_ref[0] + a1_ref[0])
    out_ref[...] = h_ref[...] + gate_ref[...] * agg


def _update(h, agg2, gate):
    grid = (N // BLK,)
    spec = pl.BlockSpec((BLK, D), lambda i: (i, 0))
    a0spec = pl.BlockSpec((1, BLK, D), lambda i: (0, i, 0))
    a1spec = pl.BlockSpec((1, BLK, D), lambda i: (1, i, 0))
    gspec = pl.BlockSpec((BLK, 1), lambda i: (i, 0))
    return pl.pallas_call(
        _update_body,
        grid=grid,
        in_specs=[spec, a0spec, a1spec, gspec],
        out_specs=spec,
        out_shape=jax.ShapeDtypeStruct((N, D), jnp.float32),
    )(h, agg2, agg2, gate)


def kernel(h, logits, old_z, edge_index, tau_1, tau_2):
    src = edge_index[0].astype(jnp.int32)
    dst = edge_index[1].astype(jnp.int32)

    p = _onehot_pred(logits)

    zc = jnp.zeros((NP_, C), jnp.float32)
    zd = jnp.zeros((NP_, D), jnp.float32)
    cnts2, agg2 = _sc_aggregate(src, dst, p, h, zc, zd)

    z, gate = _gates(cnts2, p, old_z.reshape(N, 1),
                     tau_1.reshape(1, 1), tau_2.reshape(1, 1))
    new_h = _update(h, agg2, gate)
    return (new_h, z.reshape(N))
